# baseline (speedup 1.0000x reference)
import jax
import jax.numpy as jnp
from jax import lax
from jax.experimental import pallas as pl
from jax.experimental.pallas import tpu as pltpu
from jax.experimental.pallas import tpu_sc as plsc

N = 10000
E = 320000
G = 64
D = 128

NC = 2
NS = 16
NW = NC * NS
EPW = E // NW
CH = 40
NCHUNK = EPW // CH
NB = 5
NI = 2 * NB
RPT = 624
TAIL = N - NS * RPT


def _publish(s, src_sh, dst_hbm_rows):
    rows = pl.ds(s * RPT, RPT)
    pltpu.sync_copy(src_sh.at[rows], dst_hbm_rows.at[rows])

    @pl.when(s == NS - 1)
    def _():
        tail = pl.ds(NS * RPT, TAIL)
        pltpu.sync_copy(src_sh.at[tail], dst_hbm_rows.at[tail])


def _sc_agg_body(h_hbm, src_hbm, dst_hbm, zeros_hbm, out_hbm,
                 sidx, didx, b0, b1, b2, b3, b4, acc_sh, gsem, ssem, isem):
    bufs = (b0, b1, b2, b3, b4)
    c = lax.axis_index("c")
    s = lax.axis_index("s")
    wid = c * NS + s
    eh = src_hbm.at[wid]
    dh = dst_hbm.at[wid]

    def fire_idx(q, slot):
        pltpu.async_copy(eh.at[q], sidx.at[slot], isem)
        pltpu.async_copy(dh.at[q], didx.at[slot], isem)

    def wait_idx():
        pltpu.make_async_copy(eh.at[0], sidx.at[0], isem).wait()
        pltpu.make_async_copy(dh.at[0], didx.at[0], isem).wait()

    for q in range(NI):
        fire_idx(q, q)

    pltpu.sync_copy(zeros_hbm, acc_sh.at[pl.ds(s * RPT, RPT)])

    @pl.when(s == NS - 1)
    def _():
        pltpu.sync_copy(zeros_hbm.at[pl.ds(0, TAIL)],
                        acc_sh.at[pl.ds(NS * RPT, TAIL)])

    for b in range(NB):
        wait_idx()
        pltpu.async_copy(h_hbm.at[sidx.at[b]], bufs[b], gsem)

    plsc.subcore_barrier()

    def rnd(r, carry):
        base = r * NB
        for b in range(NB):
            j = base + b
            pltpu.make_async_copy(h_hbm.at[sidx.at[0]], bufs[b],
                                  gsem).wait()
            pltpu.async_copy(bufs[b], acc_sh.at[didx.at[j % NI]], ssem,
                             add=True)
        for b in range(NB):
            j = base + b
            pltpu.make_async_copy(bufs[b], acc_sh.at[didx.at[0]],
                                  ssem).wait()
            jn2 = base + NI + b

            @pl.when(jn2 < NCHUNK)
            def _(jn2=jn2, j=j):
                fire_idx(jn2, j % NI)

            jn = base + NB + b

            @pl.when(jn < NCHUNK)
            def _(jn=jn, b=b):
                wait_idx()
                pltpu.async_copy(h_hbm.at[sidx.at[jn % NI]], bufs[b], gsem)
        return carry

    lax.fori_loop(0, NCHUNK // NB, rnd, 0)
    plsc.subcore_barrier()
    _publish(s, acc_sh, out_hbm.at[c])


def _sc_cnt_body(dst_hbm, zeros_hbm, ones_hbm, cnt_hbm,
                 dst_v, ones_v, cnt_sh, sem):
    c = lax.axis_index("c")
    s = lax.axis_index("s")
    wid = c * NS + s

    pltpu.sync_copy(zeros_hbm, cnt_sh.at[pl.ds(s * RPT, RPT)])

    @pl.when(s == NS - 1)
    def _():
        pltpu.sync_copy(zeros_hbm.at[pl.ds(0, TAIL)],
                        cnt_sh.at[pl.ds(NS * RPT, TAIL)])

    pltpu.sync_copy(ones_hbm, ones_v)
    pltpu.sync_copy(dst_hbm.at[wid], dst_v)
    plsc.subcore_barrier()

    def chunk(j, carry):
        pltpu.sync_copy(ones_v, cnt_sh.at[dst_v.at[j]], add=True)
        return carry

    lax.fori_loop(0, NCHUNK, chunk, 0)
    plsc.subcore_barrier()
    _publish(s, cnt_sh, cnt_hbm.at[c])


_MESH = plsc.VectorSubcoreMesh(core_axis_name="c", subcore_axis_name="s")

_sc_agg = pl.kernel(
    _sc_agg_body,
    out_type=(jax.ShapeDtypeStruct((NC, N, D), jnp.float32),),
    mesh=_MESH,
    scratch_types=(
        pltpu.VMEM((NI, CH), jnp.int32),
        pltpu.VMEM((NI, CH), jnp.int32),
        pltpu.VMEM((CH, D), jnp.float32),
        pltpu.VMEM((CH, D), jnp.float32),
        pltpu.VMEM((CH, D), jnp.float32),
        pltpu.VMEM((CH, D), jnp.float32),
        pltpu.VMEM((CH, D), jnp.float32),
        pltpu.VMEM_SHARED((N, D), jnp.float32),
        pltpu.SemaphoreType.DMA,
        pltpu.SemaphoreType.DMA,
        pltpu.SemaphoreType.DMA,
    ),
)

_sc_cnt = pl.kernel(
    _sc_cnt_body,
    out_type=(jax.ShapeDtypeStruct((NC, N, D), jnp.float32),),
    mesh=_MESH,
    scratch_types=(
        pltpu.VMEM((NCHUNK, CH), jnp.int32),
        pltpu.VMEM((CH, D), jnp.float32),
        pltpu.VMEM_SHARED((N, D), jnp.float32),
        pltpu.SemaphoreType.DMA,
    ),
)


BN = 2000


def _dense1_body(sums_ref, cnt_ref, h_ref, wl_ref, bl_ref, wr_ref,
                 out_ref, inv_ref):
    cnt = cnt_ref[0, :, :16] + cnt_ref[1, :, :16]
    inv = 1.0 / jnp.maximum(cnt, 1.0)
    inv_ref[...] = inv
    ssum = sums_ref[0] + sums_ref[1]
    mean = ssum * inv[:, :1]
    acc = jnp.dot(mean, wl_ref[...], preferred_element_type=jnp.float32)
    acc = acc + jnp.dot(h_ref[...], wr_ref[...],
                        preferred_element_type=jnp.float32)
    out_ref[...] = jnp.maximum(acc + bl_ref[...], 0.0)


def _dense1(sums, cnt, h, Wl, bl, Wr):
    return pl.pallas_call(
        _dense1_body,
        grid=(N // BN,),
        in_specs=[
            pl.BlockSpec((NC, BN, D), lambda i: (0, i, 0)),
            pl.BlockSpec((NC, BN, D), lambda i: (0, i, 0)),
            pl.BlockSpec((BN, D), lambda i: (i, 0)),
            pl.BlockSpec((D, D), lambda i: (0, 0)),
            pl.BlockSpec((1, D), lambda i: (0, 0)),
            pl.BlockSpec((D, D), lambda i: (0, 0)),
        ],
        out_specs=[
            pl.BlockSpec((BN, D), lambda i: (i, 0)),
            pl.BlockSpec((BN, 16), lambda i: (i, 0)),
        ],
        out_shape=[
            jax.ShapeDtypeStruct((N, D), jnp.float32),
            jax.ShapeDtypeStruct((N, 16), jnp.float32),
        ],
    )(sums, cnt, h, Wl, bl.reshape(1, D), Wr)


def _dense_body(sums_ref, inv_ref, h_ref, wl_ref, bl_ref, wr_ref, out_ref):
    ssum = sums_ref[0] + sums_ref[1]
    mean = ssum * inv_ref[:, :1]
    acc = jnp.dot(mean, wl_ref[...], preferred_element_type=jnp.float32)
    acc = acc + jnp.dot(h_ref[...], wr_ref[...],
                        preferred_element_type=jnp.float32)
    out_ref[...] = jnp.maximum(acc + bl_ref[...], 0.0)


def _dense(sums, inv16, h, Wl, bl, Wr):
    return pl.pallas_call(
        _dense_body,
        grid=(N // BN,),
        in_specs=[
            pl.BlockSpec((NC, BN, D), lambda i: (0, i, 0)),
            pl.BlockSpec((BN, 16), lambda i: (i, 0)),
            pl.BlockSpec((BN, D), lambda i: (i, 0)),
            pl.BlockSpec((D, D), lambda i: (0, 0)),
            pl.BlockSpec((1, D), lambda i: (0, 0)),
            pl.BlockSpec((D, D), lambda i: (0, 0)),
        ],
        out_specs=pl.BlockSpec((BN, D), lambda i: (i, 0)),
        out_shape=jax.ShapeDtypeStruct((N, D), jnp.float32),
    )(sums, inv16, h, Wl, bl.reshape(1, D), Wr)


def _dense_pool_body(sums_ref, inv_ref, h_ref, wl_ref, bl_ref, wr_ref,
                     batch_ref, out_ref, acc_ref, cnt_ref):
    i = pl.program_id(0)

    @pl.when(i == 0)
    def _():
        acc_ref[...] = jnp.zeros_like(acc_ref)
        cnt_ref[...] = jnp.zeros_like(cnt_ref)

    ssum = sums_ref[0] + sums_ref[1]
    mean = ssum * inv_ref[:, :1]
    acc = jnp.dot(mean, wl_ref[...], preferred_element_type=jnp.float32)
    acc = acc + jnp.dot(h_ref[...], wr_ref[...],
                        preferred_element_type=jnp.float32)
    h3 = jnp.maximum(acc + bl_ref[...], 0.0)

    b = batch_ref[0]
    gids = lax.broadcasted_iota(jnp.int32, (G, 1), 0)
    oh = (b == gids).astype(jnp.float32)
    acc_ref[...] += jnp.dot(oh, h3, preferred_element_type=jnp.float32)
    cnt_ref[...] += jnp.broadcast_to(jnp.sum(oh, axis=1, keepdims=True),
                                     (G, D))

    @pl.when(i == N // BN - 1)
    def _():
        out_ref[...] = acc_ref[...] / jnp.maximum(cnt_ref[...], 1.0)


def _dense_pool(sums, inv16, h, Wl, bl, Wr, batch3):
    return pl.pallas_call(
        _dense_pool_body,
        grid=(N // BN,),
        in_specs=[
            pl.BlockSpec((NC, BN, D), lambda i: (0, i, 0)),
            pl.BlockSpec((BN, 16), lambda i: (i, 0)),
            pl.BlockSpec((BN, D), lambda i: (i, 0)),
            pl.BlockSpec((D, D), lambda i: (0, 0)),
            pl.BlockSpec((1, D), lambda i: (0, 0)),
            pl.BlockSpec((D, D), lambda i: (0, 0)),
            pl.BlockSpec((1, 1, BN), lambda i: (i, 0, 0)),
        ],
        out_specs=pl.BlockSpec((G, D), lambda i: (0, 0)),
        out_shape=jax.ShapeDtypeStruct((G, D), jnp.float32),
        scratch_shapes=[
            pltpu.VMEM((G, D), jnp.float32),
            pltpu.VMEM((G, D), jnp.float32),
        ],
    )(sums, inv16, h, Wl, bl.reshape(1, D), Wr, batch3)


def _pool_body(h_ref, batch_ref, out_ref, acc_ref, cnt_ref):
    i = pl.program_id(0)

    @pl.when(i == 0)
    def _():
        acc_ref[...] = jnp.zeros_like(acc_ref)
        cnt_ref[...] = jnp.zeros_like(cnt_ref)

    b = batch_ref[0]
    gids = lax.broadcasted_iota(jnp.int32, (G, 1), 0)
    oh = (b == gids).astype(jnp.float32)
    acc_ref[...] += jnp.dot(oh, h_ref[...],
                            preferred_element_type=jnp.float32)
    cnt_ref[...] += jnp.broadcast_to(jnp.sum(oh, axis=1, keepdims=True),
                                     (G, D))

    @pl.when(i == N // BN - 1)
    def _():
        out_ref[...] = acc_ref[...] / jnp.maximum(cnt_ref[...], 1.0)


def _pool(h, batch3):
    return pl.pallas_call(
        _pool_body,
        grid=(N // BN,),
        in_specs=[
            pl.BlockSpec((BN, D), lambda i: (i, 0)),
            pl.BlockSpec((1, 1, BN), lambda i: (i, 0, 0)),
        ],
        out_specs=pl.BlockSpec((G, D), lambda i: (0, 0)),
        out_shape=jax.ShapeDtypeStruct((G, D), jnp.float32),
        scratch_shapes=[
            pltpu.VMEM((G, D), jnp.float32),
            pltpu.VMEM((G, D), jnp.float32),
        ],
    )(h, batch3)


def kernel(x, edge_index, batch, Wl1, bl1, Wr1, Wl2, bl2, Wr2, Wl3, bl3, Wr3):
    src = edge_index[0].reshape(NW, NCHUNK, CH)
    dst = edge_index[1].reshape(NW, NCHUNK, CH)
    zeros = jnp.zeros((RPT, D), jnp.float32)
    ones = jnp.ones((CH, D), jnp.float32)
    batch3 = batch.reshape(N // BN, 1, BN)

    cnt = _sc_cnt(dst, zeros, ones)[0]
    sums = _sc_agg(x, src, dst, zeros)[0]
    h, inv16 = _dense1(sums, cnt, x, Wl1, bl1, Wr1)
    sums = _sc_agg(h, src, dst, zeros)[0]
    h = _dense(sums, inv16, h, Wl2, bl2, Wr2)
    sums = _sc_agg(h, src, dst, zeros)[0]
    return _dense_pool(sums, inv16, h, Wl3, bl3, Wr3, batch3)


# speedup vs baseline: 1.0445x; 1.0445x over previous
import jax
import jax.numpy as jnp
from jax import lax
from jax.experimental import pallas as pl
from jax.experimental.pallas import tpu as pltpu
from jax.experimental.pallas import tpu_sc as plsc

N = 10000
E = 320000
G = 64
D = 128

NC = 2
NS = 16
NW = NC * NS
EPW = E // NW
CH = 125
NCHUNK = EPW // CH
NB = 2
NI = 2 * NB
RPT = 624
TAIL = N - NS * RPT


def _publish(s, src_sh, dst_hbm_rows):
    rows = pl.ds(s * RPT, RPT)
    pltpu.sync_copy(src_sh.at[rows], dst_hbm_rows.at[rows])

    @pl.when(s == NS - 1)
    def _():
        tail = pl.ds(NS * RPT, TAIL)
        pltpu.sync_copy(src_sh.at[tail], dst_hbm_rows.at[tail])


def _sc_agg_body(h_hbm, src_hbm, dst_hbm, zeros_hbm, out_hbm,
                 sidx, didx, b0, b1, acc_sh, gsem, ssem, isem):
    bufs = (b0, b1)
    c = lax.axis_index("c")
    s = lax.axis_index("s")
    wid = c * NS + s
    eh = src_hbm.at[wid]
    dh = dst_hbm.at[wid]

    def fire_idx(q, slot):
        pltpu.async_copy(eh.at[q], sidx.at[slot], isem)
        pltpu.async_copy(dh.at[q], didx.at[slot], isem)

    def wait_idx():
        pltpu.make_async_copy(eh.at[0], sidx.at[0], isem).wait()
        pltpu.make_async_copy(dh.at[0], didx.at[0], isem).wait()

    for q in range(NI):
        fire_idx(q, q)

    pltpu.sync_copy(zeros_hbm, acc_sh.at[pl.ds(s * RPT, RPT)])

    @pl.when(s == NS - 1)
    def _():
        pltpu.sync_copy(zeros_hbm.at[pl.ds(0, TAIL)],
                        acc_sh.at[pl.ds(NS * RPT, TAIL)])

    for b in range(NB):
        wait_idx()
        pltpu.async_copy(h_hbm.at[sidx.at[b]], bufs[b], gsem)

    plsc.subcore_barrier()

    def rnd(r, carry):
        base = r * NB
        for b in range(NB):
            j = base + b
            pltpu.make_async_copy(h_hbm.at[sidx.at[0]], bufs[b],
                                  gsem).wait()
            pltpu.async_copy(bufs[b], acc_sh.at[didx.at[j % NI]], ssem,
                             add=True)
        for b in range(NB):
            j = base + b
            pltpu.make_async_copy(bufs[b], acc_sh.at[didx.at[0]],
                                  ssem).wait()
            jn2 = base + NI + b

            @pl.when(jn2 < NCHUNK)
            def _(jn2=jn2, j=j):
                fire_idx(jn2, j % NI)

            jn = base + NB + b

            @pl.when(jn < NCHUNK)
            def _(jn=jn, b=b):
                wait_idx()
                pltpu.async_copy(h_hbm.at[sidx.at[jn % NI]], bufs[b], gsem)
        return carry

    lax.fori_loop(0, NCHUNK // NB, rnd, 0)
    plsc.subcore_barrier()
    _publish(s, acc_sh, out_hbm.at[c])


def _zero_acc(s, zeros_hbm, acc_sh):
    pltpu.sync_copy(zeros_hbm, acc_sh.at[pl.ds(s * RPT, RPT)])

    @pl.when(s == NS - 1)
    def _():
        pltpu.sync_copy(zeros_hbm.at[pl.ds(0, TAIL)],
                        acc_sh.at[pl.ds(NS * RPT, TAIL)])


def _sc_agg_cnt_body(h_hbm, src_hbm, dst_hbm, zeros_hbm, ones_hbm,
                     out_hbm, cnt_hbm,
                     sidx, didx, b0, b1, ones_v, acc_sh, gsem, ssem, isem):
    bufs = (b0, b1)
    c = lax.axis_index("c")
    s = lax.axis_index("s")
    wid = c * NS + s
    eh = src_hbm.at[wid]
    dh = dst_hbm.at[wid]

    def fire_didx(q, slot):
        pltpu.async_copy(dh.at[q], didx.at[slot], isem)

    def wait_1idx():
        pltpu.make_async_copy(dh.at[0], didx.at[0], isem).wait()

    for q in range(NI):
        fire_didx(q, q)
    pltpu.sync_copy(ones_hbm, ones_v)
    _zero_acc(s, zeros_hbm, acc_sh)
    plsc.subcore_barrier()

    def crnd(r, carry):
        base = r * NB
        for t in range(NB):
            j = base + t
            wait_1idx()
            pltpu.async_copy(ones_v, acc_sh.at[didx.at[j % NI]], ssem,
                             add=True)
        for t in range(NB):
            j = base + t
            pltpu.make_async_copy(ones_v, acc_sh.at[didx.at[0]],
                                  ssem).wait()
            jn = base + NI + t

            @pl.when(jn < NCHUNK)
            def _(jn=jn, j=j):
                fire_didx(jn, j % NI)
        return carry

    lax.fori_loop(0, NCHUNK // NB, crnd, 0)
    plsc.subcore_barrier()
    _publish(s, acc_sh, cnt_hbm.at[c])

    for q in range(NI):
        pltpu.async_copy(eh.at[q], sidx.at[q], isem)
        fire_didx(q, q)
    _zero_acc(s, zeros_hbm, acc_sh)
    for b in range(NB):
        pltpu.make_async_copy(eh.at[0], sidx.at[0], isem).wait()
        wait_1idx()
        pltpu.async_copy(h_hbm.at[sidx.at[b]], bufs[b], gsem)
    plsc.subcore_barrier()

    def rnd(r, carry):
        base = r * NB
        for b in range(NB):
            j = base + b
            pltpu.make_async_copy(h_hbm.at[sidx.at[0]], bufs[b],
                                  gsem).wait()
            pltpu.async_copy(bufs[b], acc_sh.at[didx.at[j % NI]], ssem,
                             add=True)
        for b in range(NB):
            j = base + b
            pltpu.make_async_copy(bufs[b], acc_sh.at[didx.at[0]],
                                  ssem).wait()
            jn2 = base + NI + b

            @pl.when(jn2 < NCHUNK)
            def _(jn2=jn2, j=j):
                pltpu.async_copy(eh.at[jn2], sidx.at[j % NI], isem)
                fire_didx(jn2, j % NI)

            jn = base + NB + b

            @pl.when(jn < NCHUNK)
            def _(jn=jn, b=b):
                pltpu.make_async_copy(eh.at[0], sidx.at[0], isem).wait()
                wait_1idx()
                pltpu.async_copy(h_hbm.at[sidx.at[jn % NI]], bufs[b], gsem)
        return carry

    lax.fori_loop(0, NCHUNK // NB, rnd, 0)
    plsc.subcore_barrier()
    _publish(s, acc_sh, out_hbm.at[c])


_MESH = plsc.VectorSubcoreMesh(core_axis_name="c", subcore_axis_name="s")

_sc_agg = pl.kernel(
    _sc_agg_body,
    out_type=(jax.ShapeDtypeStruct((NC, N, D), jnp.float32),),
    mesh=_MESH,
    scratch_types=(
        pltpu.VMEM((NI, CH), jnp.int32),
        pltpu.VMEM((NI, CH), jnp.int32),
        pltpu.VMEM((CH, D), jnp.float32),
        pltpu.VMEM((CH, D), jnp.float32),
        pltpu.VMEM_SHARED((N, D), jnp.float32),
        pltpu.SemaphoreType.DMA,
        pltpu.SemaphoreType.DMA,
        pltpu.SemaphoreType.DMA,
    ),
)

_sc_agg_cnt = pl.kernel(
    _sc_agg_cnt_body,
    out_type=(jax.ShapeDtypeStruct((NC, N, D), jnp.float32),
              jax.ShapeDtypeStruct((NC, N, D), jnp.float32)),
    mesh=_MESH,
    scratch_types=(
        pltpu.VMEM((NI, CH), jnp.int32),
        pltpu.VMEM((NI, CH), jnp.int32),
        pltpu.VMEM((CH, D), jnp.float32),
        pltpu.VMEM((CH, D), jnp.float32),
        pltpu.VMEM((CH, D), jnp.float32),
        pltpu.VMEM_SHARED((N, D), jnp.float32),
        pltpu.SemaphoreType.DMA,
        pltpu.SemaphoreType.DMA,
        pltpu.SemaphoreType.DMA,
    ),
)


BN = 2000


def _dense1_body(sums_ref, cnt_ref, h_ref, wl_ref, bl_ref, wr_ref,
                 out_ref, inv_ref):
    cnt = cnt_ref[0, :, :16] + cnt_ref[1, :, :16]
    inv = 1.0 / jnp.maximum(cnt, 1.0)
    inv_ref[...] = inv
    ssum = sums_ref[0] + sums_ref[1]
    mean = ssum * inv[:, :1]
    acc = jnp.dot(mean, wl_ref[...], preferred_element_type=jnp.float32)
    acc = acc + jnp.dot(h_ref[...], wr_ref[...],
                        preferred_element_type=jnp.float32)
    out_ref[...] = jnp.maximum(acc + bl_ref[...], 0.0)


def _dense1(sums, cnt, h, Wl, bl, Wr):
    return pl.pallas_call(
        _dense1_body,
        grid=(N // BN,),
        in_specs=[
            pl.BlockSpec((NC, BN, D), lambda i: (0, i, 0)),
            pl.BlockSpec((NC, BN, D), lambda i: (0, i, 0)),
            pl.BlockSpec((BN, D), lambda i: (i, 0)),
            pl.BlockSpec((D, D), lambda i: (0, 0)),
            pl.BlockSpec((1, D), lambda i: (0, 0)),
            pl.BlockSpec((D, D), lambda i: (0, 0)),
        ],
        out_specs=[
            pl.BlockSpec((BN, D), lambda i: (i, 0)),
            pl.BlockSpec((BN, 16), lambda i: (i, 0)),
        ],
        out_shape=[
            jax.ShapeDtypeStruct((N, D), jnp.float32),
            jax.ShapeDtypeStruct((N, 16), jnp.float32),
        ],
    )(sums, cnt, h, Wl, bl.reshape(1, D), Wr)


def _dense_body(sums_ref, inv_ref, h_ref, wl_ref, bl_ref, wr_ref, out_ref):
    ssum = sums_ref[0] + sums_ref[1]
    mean = ssum * inv_ref[:, :1]
    acc = jnp.dot(mean, wl_ref[...], preferred_element_type=jnp.float32)
    acc = acc + jnp.dot(h_ref[...], wr_ref[...],
                        preferred_element_type=jnp.float32)
    out_ref[...] = jnp.maximum(acc + bl_ref[...], 0.0)


def _dense(sums, inv16, h, Wl, bl, Wr):
    return pl.pallas_call(
        _dense_body,
        grid=(N // BN,),
        in_specs=[
            pl.BlockSpec((NC, BN, D), lambda i: (0, i, 0)),
            pl.BlockSpec((BN, 16), lambda i: (i, 0)),
            pl.BlockSpec((BN, D), lambda i: (i, 0)),
            pl.BlockSpec((D, D), lambda i: (0, 0)),
            pl.BlockSpec((1, D), lambda i: (0, 0)),
            pl.BlockSpec((D, D), lambda i: (0, 0)),
        ],
        out_specs=pl.BlockSpec((BN, D), lambda i: (i, 0)),
        out_shape=jax.ShapeDtypeStruct((N, D), jnp.float32),
    )(sums, inv16, h, Wl, bl.reshape(1, D), Wr)


def _dense_pool_body(sums_ref, inv_ref, h_ref, wl_ref, bl_ref, wr_ref,
                     batch_ref, out_ref, acc_ref, cnt_ref):
    i = pl.program_id(0)

    @pl.when(i == 0)
    def _():
        acc_ref[...] = jnp.zeros_like(acc_ref)
        cnt_ref[...] = jnp.zeros_like(cnt_ref)

    ssum = sums_ref[0] + sums_ref[1]
    mean = ssum * inv_ref[:, :1]
    acc = jnp.dot(mean, wl_ref[...], preferred_element_type=jnp.float32)
    acc = acc + jnp.dot(h_ref[...], wr_ref[...],
                        preferred_element_type=jnp.float32)
    h3 = jnp.maximum(acc + bl_ref[...], 0.0)

    b = batch_ref[0]
    gids = lax.broadcasted_iota(jnp.int32, (G, 1), 0)
    oh = (b == gids).astype(jnp.float32)
    acc_ref[...] += jnp.dot(oh, h3, preferred_element_type=jnp.float32)
    cnt_ref[...] += jnp.broadcast_to(jnp.sum(oh, axis=1, keepdims=True),
                                     (G, D))

    @pl.when(i == N // BN - 1)
    def _():
        out_ref[...] = acc_ref[...] / jnp.maximum(cnt_ref[...], 1.0)


def _dense_pool(sums, inv16, h, Wl, bl, Wr, batch3):
    return pl.pallas_call(
        _dense_pool_body,
        grid=(N // BN,),
        in_specs=[
            pl.BlockSpec((NC, BN, D), lambda i: (0, i, 0)),
            pl.BlockSpec((BN, 16), lambda i: (i, 0)),
            pl.BlockSpec((BN, D), lambda i: (i, 0)),
            pl.BlockSpec((D, D), lambda i: (0, 0)),
            pl.BlockSpec((1, D), lambda i: (0, 0)),
            pl.BlockSpec((D, D), lambda i: (0, 0)),
            pl.BlockSpec((1, 1, BN), lambda i: (i, 0, 0)),
        ],
        out_specs=pl.BlockSpec((G, D), lambda i: (0, 0)),
        out_shape=jax.ShapeDtypeStruct((G, D), jnp.float32),
        scratch_shapes=[
            pltpu.VMEM((G, D), jnp.float32),
            pltpu.VMEM((G, D), jnp.float32),
        ],
    )(sums, inv16, h, Wl, bl.reshape(1, D), Wr, batch3)


def _pool_body(h_ref, batch_ref, out_ref, acc_ref, cnt_ref):
    i = pl.program_id(0)

    @pl.when(i == 0)
    def _():
        acc_ref[...] = jnp.zeros_like(acc_ref)
        cnt_ref[...] = jnp.zeros_like(cnt_ref)

    b = batch_ref[0]
    gids = lax.broadcasted_iota(jnp.int32, (G, 1), 0)
    oh = (b == gids).astype(jnp.float32)
    acc_ref[...] += jnp.dot(oh, h_ref[...],
                            preferred_element_type=jnp.float32)
    cnt_ref[...] += jnp.broadcast_to(jnp.sum(oh, axis=1, keepdims=True),
                                     (G, D))

    @pl.when(i == N // BN - 1)
    def _():
        out_ref[...] = acc_ref[...] / jnp.maximum(cnt_ref[...], 1.0)


def _pool(h, batch3):
    return pl.pallas_call(
        _pool_body,
        grid=(N // BN,),
        in_specs=[
            pl.BlockSpec((BN, D), lambda i: (i, 0)),
            pl.BlockSpec((1, 1, BN), lambda i: (i, 0, 0)),
        ],
        out_specs=pl.BlockSpec((G, D), lambda i: (0, 0)),
        out_shape=jax.ShapeDtypeStruct((G, D), jnp.float32),
        scratch_shapes=[
            pltpu.VMEM((G, D), jnp.float32),
            pltpu.VMEM((G, D), jnp.float32),
        ],
    )(h, batch3)


def kernel(x, edge_index, batch, Wl1, bl1, Wr1, Wl2, bl2, Wr2, Wl3, bl3, Wr3):
    src = edge_index[0].reshape(NW, NCHUNK, CH)
    dst = edge_index[1].reshape(NW, NCHUNK, CH)
    zeros = jnp.zeros((RPT, D), jnp.float32)
    ones = jnp.ones((CH, D), jnp.float32)
    batch3 = batch.reshape(N // BN, 1, BN)

    sums, cnt = _sc_agg_cnt(x, src, dst, zeros, ones)
    h, inv16 = _dense1(sums, cnt, x, Wl1, bl1, Wr1)
    sums = _sc_agg(h, src, dst, zeros)[0]
    h = _dense(sums, inv16, h, Wl2, bl2, Wr2)
    sums = _sc_agg(h, src, dst, zeros)[0]
    return _dense_pool(sums, inv16, h, Wl3, bl3, Wr3, batch3)


# speedup vs baseline: 1.2033x; 1.1521x over previous
import jax
import jax.numpy as jnp
from jax import lax
from jax.experimental import pallas as pl
from jax.experimental.pallas import tpu as pltpu
from jax.experimental.pallas import tpu_sc as plsc

N = 10000
E = 320000
G = 64
D = 128

NC = 2
NS = 16
NW = NC * NS
EPW = E // NW
CH = 125
NCHUNK = EPW // CH
NB = 2
NI = 2 * NB
RPT = 624
TAIL = N - NS * RPT


def _publish(s, src_sh, dst_hbm_rows):
    rows = pl.ds(s * RPT, RPT)
    pltpu.sync_copy(src_sh.at[rows], dst_hbm_rows.at[rows])

    @pl.when(s == NS - 1)
    def _():
        tail = pl.ds(NS * RPT, TAIL)
        pltpu.sync_copy(src_sh.at[tail], dst_hbm_rows.at[tail])


def _sc_agg_body(h_hbm, src_hbm, dst_hbm, zeros_hbm, out_hbm,
                 sidx, didx, b0, b1, acc_sh, gsem, ssem, isem):
    bufs = (b0, b1)
    c = lax.axis_index("c")
    s = lax.axis_index("s")
    wid = c * NS + s
    eh = src_hbm.at[wid]
    dh = dst_hbm.at[wid]

    def fire_idx(q, slot):
        pltpu.async_copy(eh.at[q], sidx.at[slot], isem)
        pltpu.async_copy(dh.at[q], didx.at[slot], isem)

    def wait_idx():
        pltpu.make_async_copy(eh.at[0], sidx.at[0], isem).wait()
        pltpu.make_async_copy(dh.at[0], didx.at[0], isem).wait()

    for q in range(NI):
        fire_idx(q, q)

    pltpu.sync_copy(zeros_hbm, acc_sh.at[pl.ds(s * RPT, RPT)])

    @pl.when(s == NS - 1)
    def _():
        pltpu.sync_copy(zeros_hbm.at[pl.ds(0, TAIL)],
                        acc_sh.at[pl.ds(NS * RPT, TAIL)])

    for b in range(NB):
        wait_idx()
        pltpu.async_copy(h_hbm.at[sidx.at[b]], bufs[b], gsem)

    plsc.subcore_barrier()

    def rnd(r, carry):
        base = r * NB
        for b in range(NB):
            j = base + b
            pltpu.make_async_copy(h_hbm.at[sidx.at[0]], bufs[b],
                                  gsem).wait()
            pltpu.async_copy(bufs[b], acc_sh.at[didx.at[j % NI]], ssem,
                             add=True)
        for b in range(NB):
            j = base + b
            pltpu.make_async_copy(bufs[b], acc_sh.at[didx.at[0]],
                                  ssem).wait()
            jn2 = base + NI + b

            @pl.when(jn2 < NCHUNK)
            def _(jn2=jn2, j=j):
                fire_idx(jn2, j % NI)

            jn = base + NB + b

            @pl.when(jn < NCHUNK)
            def _(jn=jn, b=b):
                wait_idx()
                pltpu.async_copy(h_hbm.at[sidx.at[jn % NI]], bufs[b], gsem)
        return carry

    lax.fori_loop(0, NCHUNK // NB, rnd, 0)
    plsc.subcore_barrier()
    _publish(s, acc_sh, out_hbm.at[c])


_MESH = plsc.VectorSubcoreMesh(core_axis_name="c", subcore_axis_name="s")

_sc_agg = pl.kernel(
    _sc_agg_body,
    out_type=(jax.ShapeDtypeStruct((NC, N, D), jnp.float32),),
    mesh=_MESH,
    scratch_types=(
        pltpu.VMEM((NI, CH), jnp.int32),
        pltpu.VMEM((NI, CH), jnp.int32),
        pltpu.VMEM((CH, D), jnp.float32),
        pltpu.VMEM((CH, D), jnp.float32),
        pltpu.VMEM_SHARED((N, D), jnp.float32),
        pltpu.SemaphoreType.DMA,
        pltpu.SemaphoreType.DMA,
        pltpu.SemaphoreType.DMA,
    ),
)

BN = 2000

EB = 8000
NSTEP = E // EB


def _cnt_body(dst_ref, out_ref, acc_ref):
    i = pl.program_id(0)

    @pl.when(i == 0)
    def _():
        acc_ref[...] = jnp.zeros_like(acc_ref)

    d = dst_ref[0]
    a = lax.rem(d, 80)
    b = lax.div(d, 80)
    ids = lax.broadcasted_iota(jnp.int32, (D, 1), 0)
    oh_a = (a == ids).astype(jnp.bfloat16)
    oh_b = (b == ids).astype(jnp.bfloat16)
    acc_ref[...] += lax.dot_general(
        oh_a, oh_b, (((1,), (1,)), ((), ())),
        preferred_element_type=jnp.float32)

    @pl.when(i == NSTEP - 1)
    def _():
        ct = acc_ref[...].T
        out_ref[...] = 1.0 / jnp.maximum(ct[:125, :80], 1.0)


def _tc_cnt(dst4):
    return pl.pallas_call(
        _cnt_body,
        grid=(NSTEP,),
        in_specs=[pl.BlockSpec((1, 1, EB), lambda i: (i, 0, 0))],
        out_specs=pl.BlockSpec((125, 80), lambda i: (0, 0)),
        out_shape=jax.ShapeDtypeStruct((125, 80), jnp.float32),
        scratch_shapes=[pltpu.VMEM((D, D), jnp.float32)],
    )(dst4)


def _dense_body(sums_ref, inv_ref, h_ref, wl_ref, bl_ref, wr_ref, out_ref):
    ssum = sums_ref[0] + sums_ref[1]
    mean = ssum * inv_ref[...]
    acc = jnp.dot(mean, wl_ref[...], preferred_element_type=jnp.float32)
    acc = acc + jnp.dot(h_ref[...], wr_ref[...],
                        preferred_element_type=jnp.float32)
    out_ref[...] = jnp.maximum(acc + bl_ref[...], 0.0)


def _dense(sums, inv_col, h, Wl, bl, Wr):
    return pl.pallas_call(
        _dense_body,
        grid=(N // BN,),
        in_specs=[
            pl.BlockSpec((NC, BN, D), lambda i: (0, i, 0)),
            pl.BlockSpec((BN, 1), lambda i: (i, 0)),
            pl.BlockSpec((BN, D), lambda i: (i, 0)),
            pl.BlockSpec((D, D), lambda i: (0, 0)),
            pl.BlockSpec((1, D), lambda i: (0, 0)),
            pl.BlockSpec((D, D), lambda i: (0, 0)),
        ],
        out_specs=pl.BlockSpec((BN, D), lambda i: (i, 0)),
        out_shape=jax.ShapeDtypeStruct((N, D), jnp.float32),
    )(sums, inv_col, h, Wl, bl.reshape(1, D), Wr)


def _dense_pool_body(sums_ref, inv_ref, h_ref, wl_ref, bl_ref, wr_ref,
                     batch_ref, out_ref, acc_ref, cnt_ref):
    i = pl.program_id(0)

    @pl.when(i == 0)
    def _():
        acc_ref[...] = jnp.zeros_like(acc_ref)
        cnt_ref[...] = jnp.zeros_like(cnt_ref)

    ssum = sums_ref[0] + sums_ref[1]
    mean = ssum * inv_ref[...]
    acc = jnp.dot(mean, wl_ref[...], preferred_element_type=jnp.float32)
    acc = acc + jnp.dot(h_ref[...], wr_ref[...],
                        preferred_element_type=jnp.float32)
    h3 = jnp.maximum(acc + bl_ref[...], 0.0)

    b = batch_ref[0]
    gids = lax.broadcasted_iota(jnp.int32, (G, 1), 0)
    oh = (b == gids).astype(jnp.float32)
    acc_ref[...] += jnp.dot(oh, h3, preferred_element_type=jnp.float32)
    cnt_ref[...] += jnp.broadcast_to(jnp.sum(oh, axis=1, keepdims=True),
                                     (G, D))

    @pl.when(i == N // BN - 1)
    def _():
        out_ref[...] = acc_ref[...] / jnp.maximum(cnt_ref[...], 1.0)


def _dense_pool(sums, inv_col, h, Wl, bl, Wr, batch3):
    return pl.pallas_call(
        _dense_pool_body,
        grid=(N // BN,),
        in_specs=[
            pl.BlockSpec((NC, BN, D), lambda i: (0, i, 0)),
            pl.BlockSpec((BN, 1), lambda i: (i, 0)),
            pl.BlockSpec((BN, D), lambda i: (i, 0)),
            pl.BlockSpec((D, D), lambda i: (0, 0)),
            pl.BlockSpec((1, D), lambda i: (0, 0)),
            pl.BlockSpec((D, D), lambda i: (0, 0)),
            pl.BlockSpec((1, 1, BN), lambda i: (i, 0, 0)),
        ],
        out_specs=pl.BlockSpec((G, D), lambda i: (0, 0)),
        out_shape=jax.ShapeDtypeStruct((G, D), jnp.float32),
        scratch_shapes=[
            pltpu.VMEM((G, D), jnp.float32),
            pltpu.VMEM((G, D), jnp.float32),
        ],
    )(sums, inv_col, h, Wl, bl.reshape(1, D), Wr, batch3)


def _pool_body(h_ref, batch_ref, out_ref, acc_ref, cnt_ref):
    i = pl.program_id(0)

    @pl.when(i == 0)
    def _():
        acc_ref[...] = jnp.zeros_like(acc_ref)
        cnt_ref[...] = jnp.zeros_like(cnt_ref)

    b = batch_ref[0]
    gids = lax.broadcasted_iota(jnp.int32, (G, 1), 0)
    oh = (b == gids).astype(jnp.float32)
    acc_ref[...] += jnp.dot(oh, h_ref[...],
                            preferred_element_type=jnp.float32)
    cnt_ref[...] += jnp.broadcast_to(jnp.sum(oh, axis=1, keepdims=True),
                                     (G, D))

    @pl.when(i == N // BN - 1)
    def _():
        out_ref[...] = acc_ref[...] / jnp.maximum(cnt_ref[...], 1.0)


def _pool(h, batch3):
    return pl.pallas_call(
        _pool_body,
        grid=(N // BN,),
        in_specs=[
            pl.BlockSpec((BN, D), lambda i: (i, 0)),
            pl.BlockSpec((1, 1, BN), lambda i: (i, 0, 0)),
        ],
        out_specs=pl.BlockSpec((G, D), lambda i: (0, 0)),
        out_shape=jax.ShapeDtypeStruct((G, D), jnp.float32),
        scratch_shapes=[
            pltpu.VMEM((G, D), jnp.float32),
            pltpu.VMEM((G, D), jnp.float32),
        ],
    )(h, batch3)


def kernel(x, edge_index, batch, Wl1, bl1, Wr1, Wl2, bl2, Wr2, Wl3, bl3, Wr3):
    src = edge_index[0].reshape(NW, NCHUNK, CH)
    dst = edge_index[1].reshape(NW, NCHUNK, CH)
    dst4 = edge_index[1].reshape(NSTEP, 1, EB)
    zeros = jnp.zeros((RPT, D), jnp.float32)
    batch3 = batch.reshape(N // BN, 1, BN)

    sums = _sc_agg(x, src, dst, zeros)[0]
    inv_col = _tc_cnt(dst4).reshape(N, 1)
    h = _dense(sums, inv_col, x, Wl1, bl1, Wr1)
    sums = _sc_agg(h, src, dst, zeros)[0]
    h = _dense(sums, inv_col, h, Wl2, bl2, Wr2)
    sums = _sc_agg(h, src, dst, zeros)[0]
    return _dense_pool(sums, inv_col, h, Wl3, bl3, Wr3, batch3)


# speedup vs baseline: 1.2155x; 1.0102x over previous
import jax
import jax.numpy as jnp
from jax import lax
from jax.experimental import pallas as pl
from jax.experimental.pallas import tpu as pltpu
from jax.experimental.pallas import tpu_sc as plsc

N = 10000
E = 320000
G = 64
D = 128

NC = 2
NS = 16
NW = NC * NS
EPW = E // NW
CH = 125
NCHUNK = EPW // CH
NB = 2
NI = 2 * NB
RPT = 624
TAIL = N - NS * RPT


def _publish(s, src_sh, dst_hbm_rows):
    rows = pl.ds(s * RPT, RPT)
    pltpu.sync_copy(src_sh.at[rows], dst_hbm_rows.at[rows])

    @pl.when(s == NS - 1)
    def _():
        tail = pl.ds(NS * RPT, TAIL)
        pltpu.sync_copy(src_sh.at[tail], dst_hbm_rows.at[tail])


def _sc_agg_body(h_hbm, src_hbm, dst_hbm, zeros_hbm, out_hbm,
                 sidx, didx, b0, b1, acc_sh, gsem, ssem, isem):
    bufs = (b0, b1)
    c = lax.axis_index("c")
    s = lax.axis_index("s")
    wid = c * NS + s
    eh = src_hbm.at[wid]
    dh = dst_hbm.at[wid]

    def fire_idx(q, slot):
        pltpu.async_copy(eh.at[q], sidx.at[slot], isem)
        pltpu.async_copy(dh.at[q], didx.at[slot], isem)

    def wait_idx():
        pltpu.make_async_copy(eh.at[0], sidx.at[0], isem).wait()
        pltpu.make_async_copy(dh.at[0], didx.at[0], isem).wait()

    for q in range(NI):
        fire_idx(q, q)

    pltpu.sync_copy(zeros_hbm, acc_sh.at[pl.ds(s * RPT, RPT)])

    @pl.when(s == NS - 1)
    def _():
        pltpu.sync_copy(zeros_hbm.at[pl.ds(0, TAIL)],
                        acc_sh.at[pl.ds(NS * RPT, TAIL)])

    for b in range(NB):
        wait_idx()
        pltpu.async_copy(h_hbm.at[sidx.at[b]], bufs[b], gsem)

    plsc.subcore_barrier()

    def rnd(r, carry):
        base = r * NB
        for b in range(NB):
            j = base + b
            pltpu.make_async_copy(h_hbm.at[sidx.at[0]], bufs[b],
                                  gsem).wait()
            pltpu.async_copy(bufs[b], acc_sh.at[didx.at[j % NI]], ssem,
                             add=True)
        for b in range(NB):
            j = base + b
            pltpu.make_async_copy(bufs[b], acc_sh.at[didx.at[0]],
                                  ssem).wait()
            jn2 = base + NI + b

            @pl.when(jn2 < NCHUNK)
            def _(jn2=jn2, j=j):
                fire_idx(jn2, j % NI)

            jn = base + NB + b

            @pl.when(jn < NCHUNK)
            def _(jn=jn, b=b):
                wait_idx()
                pltpu.async_copy(h_hbm.at[sidx.at[jn % NI]], bufs[b], gsem)
        return carry

    lax.fori_loop(0, NCHUNK // NB, rnd, 0)
    plsc.subcore_barrier()
    _publish(s, acc_sh, out_hbm.at[c])


_MESH = plsc.VectorSubcoreMesh(core_axis_name="c", subcore_axis_name="s")

_sc_agg = pl.kernel(
    _sc_agg_body,
    out_type=(jax.ShapeDtypeStruct((NC, N, D), jnp.float32),),
    mesh=_MESH,
    scratch_types=(
        pltpu.VMEM((NI, CH), jnp.int32),
        pltpu.VMEM((NI, CH), jnp.int32),
        pltpu.VMEM((CH, D), jnp.float32),
        pltpu.VMEM((CH, D), jnp.float32),
        pltpu.VMEM_SHARED((N, D), jnp.float32),
        pltpu.SemaphoreType.DMA,
        pltpu.SemaphoreType.DMA,
        pltpu.SemaphoreType.DMA,
    ),
)

BN = 5000

EB = 8000
NSTEP = E // EB


def _cnt_body(dst_ref, out_ref, acc_ref):
    i = pl.program_id(0)

    @pl.when(i == 0)
    def _():
        acc_ref[...] = jnp.zeros_like(acc_ref)

    d = dst_ref[0]
    a = lax.rem(d, 80)
    b = lax.div(d, 80)
    ids = lax.broadcasted_iota(jnp.int32, (D, 1), 0)
    oh_a = (a == ids).astype(jnp.bfloat16)
    oh_b = (b == ids).astype(jnp.bfloat16)
    acc_ref[...] += lax.dot_general(
        oh_a, oh_b, (((1,), (1,)), ((), ())),
        preferred_element_type=jnp.float32)

    @pl.when(i == NSTEP - 1)
    def _():
        ct = acc_ref[...].T
        out_ref[...] = 1.0 / jnp.maximum(ct[:125, :80], 1.0)


def _tc_cnt(dst4):
    return pl.pallas_call(
        _cnt_body,
        grid=(NSTEP,),
        in_specs=[pl.BlockSpec((1, 1, EB), lambda i: (i, 0, 0))],
        out_specs=pl.BlockSpec((125, 80), lambda i: (0, 0)),
        out_shape=jax.ShapeDtypeStruct((125, 80), jnp.float32),
        scratch_shapes=[pltpu.VMEM((D, D), jnp.float32)],
    )(dst4)


def _dense_body(sums_ref, inv_ref, h_ref, wl_ref, bl_ref, wr_ref, out_ref):
    ssum = sums_ref[0] + sums_ref[1]
    mean = ssum * inv_ref[...]
    acc = jnp.dot(mean, wl_ref[...], preferred_element_type=jnp.float32)
    acc = acc + jnp.dot(h_ref[...], wr_ref[...],
                        preferred_element_type=jnp.float32)
    out_ref[...] = jnp.maximum(acc + bl_ref[...], 0.0)


def _dense(sums, inv_col, h, Wl, bl, Wr):
    return pl.pallas_call(
        _dense_body,
        grid=(N // BN,),
        in_specs=[
            pl.BlockSpec((NC, BN, D), lambda i: (0, i, 0)),
            pl.BlockSpec((BN, 1), lambda i: (i, 0)),
            pl.BlockSpec((BN, D), lambda i: (i, 0)),
            pl.BlockSpec((D, D), lambda i: (0, 0)),
            pl.BlockSpec((1, D), lambda i: (0, 0)),
            pl.BlockSpec((D, D), lambda i: (0, 0)),
        ],
        out_specs=pl.BlockSpec((BN, D), lambda i: (i, 0)),
        out_shape=jax.ShapeDtypeStruct((N, D), jnp.float32),
    )(sums, inv_col, h, Wl, bl.reshape(1, D), Wr)


def _dense_pool_body(sums_ref, inv_ref, h_ref, wl_ref, bl_ref, wr_ref,
                     batch_ref, out_ref, acc_ref, cnt_ref):
    i = pl.program_id(0)

    @pl.when(i == 0)
    def _():
        acc_ref[...] = jnp.zeros_like(acc_ref)
        cnt_ref[...] = jnp.zeros_like(cnt_ref)

    ssum = sums_ref[0] + sums_ref[1]
    mean = ssum * inv_ref[...]
    acc = jnp.dot(mean, wl_ref[...], preferred_element_type=jnp.float32)
    acc = acc + jnp.dot(h_ref[...], wr_ref[...],
                        preferred_element_type=jnp.float32)
    h3 = jnp.maximum(acc + bl_ref[...], 0.0)

    b = batch_ref[0]
    gids = lax.broadcasted_iota(jnp.int32, (G, 1), 0)
    oh = (b == gids).astype(jnp.float32)
    acc_ref[...] += jnp.dot(oh, h3, preferred_element_type=jnp.float32)
    cnt_ref[...] += jnp.broadcast_to(jnp.sum(oh, axis=1, keepdims=True),
                                     (G, D))

    @pl.when(i == N // BN - 1)
    def _():
        out_ref[...] = acc_ref[...] / jnp.maximum(cnt_ref[...], 1.0)


def _dense_pool(sums, inv_col, h, Wl, bl, Wr, batch3):
    return pl.pallas_call(
        _dense_pool_body,
        grid=(N // BN,),
        in_specs=[
            pl.BlockSpec((NC, BN, D), lambda i: (0, i, 0)),
            pl.BlockSpec((BN, 1), lambda i: (i, 0)),
            pl.BlockSpec((BN, D), lambda i: (i, 0)),
            pl.BlockSpec((D, D), lambda i: (0, 0)),
            pl.BlockSpec((1, D), lambda i: (0, 0)),
            pl.BlockSpec((D, D), lambda i: (0, 0)),
            pl.BlockSpec((1, 1, BN), lambda i: (i, 0, 0)),
        ],
        out_specs=pl.BlockSpec((G, D), lambda i: (0, 0)),
        out_shape=jax.ShapeDtypeStruct((G, D), jnp.float32),
        scratch_shapes=[
            pltpu.VMEM((G, D), jnp.float32),
            pltpu.VMEM((G, D), jnp.float32),
        ],
    )(sums, inv_col, h, Wl, bl.reshape(1, D), Wr, batch3)


def _pool_body(h_ref, batch_ref, out_ref, acc_ref, cnt_ref):
    i = pl.program_id(0)

    @pl.when(i == 0)
    def _():
        acc_ref[...] = jnp.zeros_like(acc_ref)
        cnt_ref[...] = jnp.zeros_like(cnt_ref)

    b = batch_ref[0]
    gids = lax.broadcasted_iota(jnp.int32, (G, 1), 0)
    oh = (b == gids).astype(jnp.float32)
    acc_ref[...] += jnp.dot(oh, h_ref[...],
                            preferred_element_type=jnp.float32)
    cnt_ref[...] += jnp.broadcast_to(jnp.sum(oh, axis=1, keepdims=True),
                                     (G, D))

    @pl.when(i == N // BN - 1)
    def _():
        out_ref[...] = acc_ref[...] / jnp.maximum(cnt_ref[...], 1.0)


def _pool(h, batch3):
    return pl.pallas_call(
        _pool_body,
        grid=(N // BN,),
        in_specs=[
            pl.BlockSpec((BN, D), lambda i: (i, 0)),
            pl.BlockSpec((1, 1, BN), lambda i: (i, 0, 0)),
        ],
        out_specs=pl.BlockSpec((G, D), lambda i: (0, 0)),
        out_shape=jax.ShapeDtypeStruct((G, D), jnp.float32),
        scratch_shapes=[
            pltpu.VMEM((G, D), jnp.float32),
            pltpu.VMEM((G, D), jnp.float32),
        ],
    )(h, batch3)


def kernel(x, edge_index, batch, Wl1, bl1, Wr1, Wl2, bl2, Wr2, Wl3, bl3, Wr3):
    src = edge_index[0].reshape(NW, NCHUNK, CH)
    dst = edge_index[1].reshape(NW, NCHUNK, CH)
    dst4 = edge_index[1].reshape(NSTEP, 1, EB)
    zeros = jnp.zeros((RPT, D), jnp.float32)
    batch3 = batch.reshape(N // BN, 1, BN)

    sums = _sc_agg(x, src, dst, zeros)[0]
    inv_col = _tc_cnt(dst4).reshape(N, 1)
    h = _dense(sums, inv_col, x, Wl1, bl1, Wr1)
    sums = _sc_agg(h, src, dst, zeros)[0]
    h = _dense(sums, inv_col, h, Wl2, bl2, Wr2)
    sums = _sc_agg(h, src, dst, zeros)[0]
    return _dense_pool(sums, inv_col, h, Wl3, bl3, Wr3, batch3)


# speedup vs baseline: 1.2156x; 1.0001x over previous
import jax
import jax.numpy as jnp
from jax import lax
from jax.experimental import pallas as pl
from jax.experimental.pallas import tpu as pltpu
from jax.experimental.pallas import tpu_sc as plsc

N = 10000
E = 320000
G = 64
D = 128

NC = 2
NS = 16
NW = NC * NS
EPW = E // NW
CH = 125
NCHUNK = EPW // CH
NB = 2
NI = 2 * NB
RPT = 624
TAIL = N - NS * RPT


def _publish(s, src_sh, dst_hbm_rows):
    rows = pl.ds(s * RPT, RPT)
    pltpu.sync_copy(src_sh.at[rows], dst_hbm_rows.at[rows])

    @pl.when(s == NS - 1)
    def _():
        tail = pl.ds(NS * RPT, TAIL)
        pltpu.sync_copy(src_sh.at[tail], dst_hbm_rows.at[tail])


def _sc_agg_body(h_hbm, src_hbm, dst_hbm, zeros_hbm, out_hbm,
                 sidx, didx, b0, b1, acc_sh, gsem, ssem, isem):
    bufs = (b0, b1)
    c = lax.axis_index("c")
    s = lax.axis_index("s")
    wid = c * NS + s
    eh = src_hbm.at[wid]
    dh = dst_hbm.at[wid]

    def fire_idx(q, slot):
        pltpu.async_copy(eh.at[q], sidx.at[slot], isem)
        pltpu.async_copy(dh.at[q], didx.at[slot], isem)

    def wait_idx():
        pltpu.make_async_copy(eh.at[0], sidx.at[0], isem).wait()
        pltpu.make_async_copy(dh.at[0], didx.at[0], isem).wait()

    for q in range(NI):
        fire_idx(q, q)

    pltpu.sync_copy(zeros_hbm, acc_sh.at[pl.ds(s * RPT, RPT)])

    @pl.when(s == NS - 1)
    def _():
        pltpu.sync_copy(zeros_hbm.at[pl.ds(0, TAIL)],
                        acc_sh.at[pl.ds(NS * RPT, TAIL)])

    for b in range(NB):
        wait_idx()
        pltpu.async_copy(h_hbm.at[sidx.at[b]], bufs[b], gsem)

    plsc.subcore_barrier()

    def rnd(r, carry):
        base = r * NB
        for b in range(NB):
            j = base + b
            pltpu.make_async_copy(h_hbm.at[sidx.at[0]], bufs[b],
                                  gsem).wait()
            pltpu.async_copy(bufs[b], acc_sh.at[didx.at[j % NI]], ssem,
                             add=True)
        for b in range(NB):
            j = base + b
            pltpu.make_async_copy(bufs[b], acc_sh.at[didx.at[0]],
                                  ssem).wait()
            jn2 = base + NI + b

            @pl.when(jn2 < NCHUNK)
            def _(jn2=jn2, j=j):
                fire_idx(jn2, j % NI)

            jn = base + NB + b

            @pl.when(jn < NCHUNK)
            def _(jn=jn, b=b):
                wait_idx()
                pltpu.async_copy(h_hbm.at[sidx.at[jn % NI]], bufs[b], gsem)
        return carry

    lax.fori_loop(0, NCHUNK // NB, rnd, 0)
    plsc.subcore_barrier()
    _publish(s, acc_sh, out_hbm.at[c])


_MESH = plsc.VectorSubcoreMesh(core_axis_name="c", subcore_axis_name="s")

_sc_agg = pl.kernel(
    _sc_agg_body,
    out_type=(jax.ShapeDtypeStruct((NC, N, D), jnp.float32),),
    mesh=_MESH,
    scratch_types=(
        pltpu.VMEM((NI, CH), jnp.int32),
        pltpu.VMEM((NI, CH), jnp.int32),
        pltpu.VMEM((CH, D), jnp.float32),
        pltpu.VMEM((CH, D), jnp.float32),
        pltpu.VMEM_SHARED((N, D), jnp.float32),
        pltpu.SemaphoreType.DMA,
        pltpu.SemaphoreType.DMA,
        pltpu.SemaphoreType.DMA,
    ),
)

BN = 5000

EB = 8000
NSTEP = E // EB


def _cnt_body(dst_ref, out_ref, acc_ref):
    i = pl.program_id(0)

    @pl.when(i == 0)
    def _():
        acc_ref[...] = jnp.zeros_like(acc_ref)

    d = dst_ref[0]
    a = lax.rem(d, 80)
    b = lax.div(d, 80)
    ids = lax.broadcasted_iota(jnp.int32, (D, 1), 0)
    oh_a = (a == ids).astype(jnp.bfloat16)
    oh_b = (b == ids).astype(jnp.bfloat16)
    acc_ref[...] += lax.dot_general(
        oh_a, oh_b, (((1,), (1,)), ((), ())),
        preferred_element_type=jnp.float32)

    @pl.when(i == NSTEP - 1)
    def _():
        ct = acc_ref[...].T
        out_ref[...] = 1.0 / jnp.maximum(ct[:125, :80], 1.0)


def _tc_cnt(dst4):
    return pl.pallas_call(
        _cnt_body,
        grid=(NSTEP,),
        in_specs=[pl.BlockSpec((1, 1, EB), lambda i: (i, 0, 0))],
        out_specs=pl.BlockSpec((125, 80), lambda i: (0, 0)),
        out_shape=jax.ShapeDtypeStruct((125, 80), jnp.float32),
        scratch_shapes=[pltpu.VMEM((D, D), jnp.float32)],
    )(dst4)


def _dense_body(sums_ref, inv_ref, h_ref, wl_ref, bl_ref, wr_ref, out_ref):
    ssum = sums_ref[0] + sums_ref[1]
    mean = ssum * inv_ref[...]
    acc = jnp.dot(mean, wl_ref[...], preferred_element_type=jnp.float32)
    acc = acc + jnp.dot(h_ref[...], wr_ref[...],
                        preferred_element_type=jnp.float32)
    out_ref[...] = jnp.maximum(acc + bl_ref[...], 0.0)


def _dense(sums, inv_col, h, Wl, bl, Wr):
    return pl.pallas_call(
        _dense_body,
        grid=(N // BN,),
        in_specs=[
            pl.BlockSpec((NC, BN, D), lambda i: (0, i, 0)),
            pl.BlockSpec((BN, 1), lambda i: (i, 0)),
            pl.BlockSpec((BN, D), lambda i: (i, 0)),
            pl.BlockSpec((D, D), lambda i: (0, 0)),
            pl.BlockSpec((1, D), lambda i: (0, 0)),
            pl.BlockSpec((D, D), lambda i: (0, 0)),
        ],
        out_specs=pl.BlockSpec((BN, D), lambda i: (i, 0)),
        out_shape=jax.ShapeDtypeStruct((N, D), jnp.float32),
    )(sums, inv_col, h, Wl, bl.reshape(1, D), Wr)


def _dense_pool_body(sums_ref, inv_ref, h_ref, wl_ref, bl_ref, wr_ref,
                     batch_ref, out_ref, acc_ref, cnt_ref):
    i = pl.program_id(0)

    @pl.when(i == 0)
    def _():
        acc_ref[...] = jnp.zeros_like(acc_ref)
        cnt_ref[...] = jnp.zeros_like(cnt_ref)

    ssum = sums_ref[0] + sums_ref[1]
    mean = ssum * inv_ref[...]
    acc = jnp.dot(mean, wl_ref[...], preferred_element_type=jnp.float32)
    acc = acc + jnp.dot(h_ref[...], wr_ref[...],
                        preferred_element_type=jnp.float32)
    h3 = jnp.maximum(acc + bl_ref[...], 0.0)

    b = batch_ref[0]
    gids = lax.broadcasted_iota(jnp.int32, (G, 1), 0)
    oh = (b == gids).astype(jnp.float32)
    acc_ref[...] += jnp.dot(oh, h3, preferred_element_type=jnp.float32)
    cnt_ref[...] += jnp.broadcast_to(jnp.sum(oh, axis=1, keepdims=True),
                                     (G, D))

    @pl.when(i == N // BN - 1)
    def _():
        out_ref[...] = acc_ref[...] / jnp.maximum(cnt_ref[...], 1.0)


def _dense_pool(sums, inv_col, h, Wl, bl, Wr, batch3):
    return pl.pallas_call(
        _dense_pool_body,
        grid=(N // BN,),
        in_specs=[
            pl.BlockSpec((NC, BN, D), lambda i: (0, i, 0)),
            pl.BlockSpec((BN, 1), lambda i: (i, 0)),
            pl.BlockSpec((BN, D), lambda i: (i, 0)),
            pl.BlockSpec((D, D), lambda i: (0, 0)),
            pl.BlockSpec((1, D), lambda i: (0, 0)),
            pl.BlockSpec((D, D), lambda i: (0, 0)),
            pl.BlockSpec((1, 1, BN), lambda i: (i, 0, 0)),
        ],
        out_specs=pl.BlockSpec((G, D), lambda i: (0, 0)),
        out_shape=jax.ShapeDtypeStruct((G, D), jnp.float32),
        scratch_shapes=[
            pltpu.VMEM((G, D), jnp.float32),
            pltpu.VMEM((G, D), jnp.float32),
        ],
    )(sums, inv_col, h, Wl, bl.reshape(1, D), Wr, batch3)


def kernel(x, edge_index, batch, Wl1, bl1, Wr1, Wl2, bl2, Wr2, Wl3, bl3, Wr3):
    src = edge_index[0].reshape(NW, NCHUNK, CH)
    dst = edge_index[1].reshape(NW, NCHUNK, CH)
    dst4 = edge_index[1].reshape(NSTEP, 1, EB)
    zeros = jnp.zeros((RPT, D), jnp.float32)
    batch3 = batch.reshape(N // BN, 1, BN)

    sums = _sc_agg(x, src, dst, zeros)[0]
    inv_col = _tc_cnt(dst4).reshape(N, 1)
    h = _dense(sums, inv_col, x, Wl1, bl1, Wr1)
    sums = _sc_agg(h, src, dst, zeros)[0]
    h = _dense(sums, inv_col, h, Wl2, bl2, Wr2)
    sums = _sc_agg(h, src, dst, zeros)[0]
    return _dense_pool(sums, inv_col, h, Wl3, bl3, Wr3, batch3)
